# Initial kernel scaffold; baseline (speedup 1.0000x reference)
import jax
import jax.numpy as jnp
from jax.experimental import pallas as pl


def kernel(nfeat, degs, row_node_idx, perm_node_idx, W, bias, W0, b0, W1, b1, Wf, bf):
    raise NotImplementedError("write your pallas kernel here")


# speedup vs baseline: 4.6170x; 4.6170x over previous
import functools

import jax
import jax.numpy as jnp
from jax import lax
from jax.experimental import pallas as pl
from jax.experimental.pallas import tpu as pltpu
from jax.experimental.pallas import tpu_sc as plsc

N = 10000
P = 40000
L = 16
IN_DIM = 128
HID = 128
OUT_DIM = 64

NC = 2
NS = 16
NW = NC * NS
PPAD = 40960
PERMS_PER_TILE = PPAD // NW
CHUNK = 16
NCHUNK = PERMS_PER_TILE // CHUNK
ROWS = CHUNK * L
NPOOL = 10240
ZROWS = NPOOL // NS


def _matmul_body(a_ref, b_ref, o_ref):
    o_ref[...] = jnp.dot(a_ref[...], b_ref[...],
                         preferred_element_type=jnp.float32)


def _compute_g(nfeat, wt):
    bm = 400
    return pl.pallas_call(
        _matmul_body,
        grid=(N // bm,),
        in_specs=[
            pl.BlockSpec((bm, IN_DIM), lambda i: (i, 0)),
            pl.BlockSpec((IN_DIM, L * HID), lambda i: (0, 0)),
        ],
        out_specs=pl.BlockSpec((bm, L * HID), lambda i: (i, 0)),
        out_shape=jax.ShapeDtypeStruct((N, L * HID), jnp.float32),
    )(nfeat, wt)


_MESH = plsc.VectorSubcoreMesh(core_axis_name="c", subcore_axis_name="s")


@functools.partial(
    pl.kernel,
    out_type=jax.ShapeDtypeStruct((NC, NPOOL, HID), jnp.float32),
    mesh=_MESH,
    scratch_types=[
        pltpu.VMEM((NCHUNK, CHUNK), jnp.int32),
        pltpu.VMEM((ROWS,), jnp.int32),
        pltpu.VMEM((2, 128), jnp.int32),
        pltpu.VMEM((ROWS, HID), jnp.float32),
        pltpu.VMEM((CHUNK, HID), jnp.float32),
        pltpu.VMEM((HID,), jnp.float32),
        pltpu.VMEM_SHARED((NPOOL, HID), jnp.float32),
        pltpu.SemaphoreType.DMA,
        pltpu.SemaphoreType.DMA,
    ],
)
def _sc_pool(g_hbm, idx_hbm, pni_hbm, bias_hbm, out_hbm,
             pni_v, idx_v, flat_v, rows_v, hperm_v, bias_v, pool_sh,
             sem0, sem1):
    cid = lax.axis_index("c")
    sid = lax.axis_index("s")
    wid = cid * NS + sid

    zeros16 = jnp.zeros((16,), jnp.float32)

    def zero_row(i, _):
        for c in range(HID // 16):
            rows_v[i, pl.ds(c * 16, 16)] = zeros16
        return 0

    lax.fori_loop(0, ROWS, zero_row, 0)
    zbase = sid * ZROWS
    pltpu.sync_copy(rows_v, pool_sh.at[pl.ds(zbase, ROWS)])
    pltpu.sync_copy(rows_v, pool_sh.at[pl.ds(zbase + ROWS, ROWS)])
    pltpu.sync_copy(rows_v.at[pl.ds(0, ZROWS - 2 * ROWS)],
                    pool_sh.at[pl.ds(zbase + 2 * ROWS, ZROWS - 2 * ROWS)])
    plsc.subcore_barrier()

    pltpu.sync_copy(bias_hbm, bias_v)
    pltpu.sync_copy(pni_hbm.at[wid], pni_v)
    lanes = lax.iota(jnp.int32, 16)

    def chunk_body(it, _):
        pltpu.sync_copy(idx_hbm.at[wid, it], idx_v)
        for k in range(ROWS // 16):
            v = idx_v[pl.ds(k * 16, 16)]
            flat_v[k // 8, pl.ds((k % 8) * 16, 16)] = v * L + lanes
        cp0 = pltpu.make_async_copy(g_hbm.at[flat_v.at[0]],
                                    rows_v.at[pl.ds(0, 128)], sem0)
        cp1 = pltpu.make_async_copy(g_hbm.at[flat_v.at[1]],
                                    rows_v.at[pl.ds(128, 128)], sem1)
        cp0.start()
        cp1.start()
        cp0.wait()
        cp1.wait()

        def perm_body(p, _):
            r0 = p * L
            for c in range(HID // 16):
                acc = rows_v[r0, pl.ds(c * 16, 16)]
                for a in range(1, L):
                    acc = acc + rows_v[r0 + a, pl.ds(c * 16, 16)]
                acc = jnp.maximum(acc + bias_v[pl.ds(c * 16, 16)], 0.0)
                hperm_v[p, pl.ds(c * 16, 16)] = acc
            return 0

        lax.fori_loop(0, CHUNK, perm_body, 0)
        pltpu.sync_copy(hperm_v, pool_sh.at[pni_v.at[it]], add=True)
        return 0

    lax.fori_loop(0, NCHUNK, chunk_body, 0)

    plsc.subcore_barrier()
    pltpu.sync_copy(pool_sh.at[pl.ds(zbase, ZROWS)],
                    out_hbm.at[cid, pl.ds(zbase, ZROWS)])


def _final_body(pooled_ref, degs_ref, w0_ref, b0_ref, w1_ref, b1_ref,
                wf_ref, bf_ref, o_ref):
    pooled = pooled_ref[0] + pooled_ref[1]
    f = jnp.maximum(degs_ref[...] * w0_ref[...] + b0_ref[...], 0.0)
    f = jnp.dot(f, w1_ref[...], preferred_element_type=jnp.float32) + b1_ref[...]
    node_h = jnp.maximum(pooled * f, 0.0)
    o_ref[...] = jnp.dot(node_h, wf_ref[...],
                         preferred_element_type=jnp.float32) + bf_ref[...]


def _final(pooled2, degs, w0t, b0, w1t, b1, wft, bf):
    bm = 400
    return pl.pallas_call(
        _final_body,
        grid=(N // bm,),
        in_specs=[
            pl.BlockSpec((NC, bm, HID), lambda i: (0, i, 0)),
            pl.BlockSpec((bm, 1), lambda i: (i, 0)),
            pl.BlockSpec((1, 2 * HID), lambda i: (0, 0)),
            pl.BlockSpec((1, 2 * HID), lambda i: (0, 0)),
            pl.BlockSpec((2 * HID, HID), lambda i: (0, 0)),
            pl.BlockSpec((1, HID), lambda i: (0, 0)),
            pl.BlockSpec((HID, OUT_DIM), lambda i: (0, 0)),
            pl.BlockSpec((1, OUT_DIM), lambda i: (0, 0)),
        ],
        out_specs=pl.BlockSpec((bm, OUT_DIM), lambda i: (i, 0)),
        out_shape=jax.ShapeDtypeStruct((N, OUT_DIM), jnp.float32),
    )(pooled2, degs, w0t, b0, w1t, b1, wft, bf)


def kernel(nfeat, degs, row_node_idx, perm_node_idx, W, bias, W0, b0, W1, b1,
           Wf, bf):
    wt = jnp.transpose(W, (0, 2, 1)).reshape(IN_DIM, L * HID)
    g = _compute_g(nfeat, wt).reshape(N * L, HID)

    idx = row_node_idx.astype(jnp.int32)
    pni = perm_node_idx.astype(jnp.int32)
    idx_p = jnp.concatenate(
        [idx, jnp.zeros(((PPAD - P) * L,), jnp.int32)]).reshape(NW, NCHUNK, ROWS)
    pni_p = jnp.concatenate(
        [pni, jnp.full((PPAD - P,), N, jnp.int32)]).reshape(NW, NCHUNK, CHUNK)

    pooled2 = _sc_pool(g, idx_p, pni_p, bias.reshape(HID))

    return _final(pooled2, degs.reshape(N, 1),
                  W0.reshape(1, 2 * HID), b0.reshape(1, 2 * HID),
                  jnp.transpose(W1), b1.reshape(1, HID),
                  jnp.transpose(Wf), bf.reshape(1, OUT_DIM))
